# baseline (speedup 1.0000x reference)
import functools

import jax
import jax.numpy as jnp
from jax import lax
from jax.experimental import pallas as pl
from jax.experimental.pallas import tpu as pltpu
from jax.experimental.pallas import tpu_sc as plsc

VOCAB = 32128
DIM = 896
DIM2 = DIM // 2
DIMP2 = 512
DIMP = 2 * DIMP2
BATCH = 4096
SEQ = 128
EPS = 1.1920928955078125e-07

LANES = 16
NUM_WORKERS = 32
BPW = BATCH // NUM_WORKERS
HALF = BPW
CHUNK = 64
NBUF = SEQ // CHUNK
GROUPS = DIM2 // LANES


def _rne_bf16_bits(y):
    b = lax.bitcast_convert_type(y, jnp.int32)
    return lax.shift_right_logical(
        b + 0x7FFF + (lax.shift_right_logical(b, 16) & 1), 16)


def _proj_norm_body(nw_ref, emb_ref, w_ref, out_ref):
    x = emb_ref[...]
    y = lax.dot_general(x, w_ref[...], (((1,), (1,)), ((), ())),
                        preferred_element_type=jnp.float32)
    ms = jnp.mean(y * y, axis=1, keepdims=True)
    yn = y * lax.rsqrt(ms + EPS) * nw_ref[...]
    lo = _rne_bf16_bits(yn[:, :DIM2]) & 0xFFFF
    hi = lax.shift_left(_rne_bf16_bits(yn[:, DIM2:]), 16)
    out_ref[:, :DIM2] = lo | hi


def _projected_table(embed_table, proj_w, norm_w):
    blk = 2048
    return pl.pallas_call(
        _proj_norm_body,
        grid=(pl.cdiv(VOCAB, blk),),
        in_specs=[
            pl.BlockSpec((1, DIM), lambda i: (0, 0)),
            pl.BlockSpec((blk, DIM), lambda i: (i, 0)),
            pl.BlockSpec((DIM, DIM), lambda i: (0, 0)),
        ],
        out_specs=pl.BlockSpec((blk, DIMP2), lambda i: (i, 0)),
        out_shape=jax.ShapeDtypeStruct((VOCAB, DIMP2), jnp.int32),
    )(norm_w.reshape(1, DIM), embed_table, proj_w)


def _pool_body(tok_hbm, ntab_hbm, out_hbm,
               tok_v, bufs, acc_v, pair_v, sems, osem):
    wid = lax.axis_index("s") * 2 + lax.axis_index("c")
    base = wid * BPW

    def gstart(row, h, buf, sem):
        idx = tok_v.at[row, pl.ds(h * CHUNK, CHUNK)]
        pltpu.make_async_copy(ntab_hbm.at[idx], buf, sem).start()

    def gwait(buf, sem):
        idx = tok_v.at[0, pl.ds(0, CHUNK)]
        pltpu.make_async_copy(ntab_hbm.at[idx], buf, sem).wait()

    def accum(buf, slot, h):
        def group_body(c, _):
            sl = pl.ds(c * LANES, LANES)

            z = jnp.zeros((LANES,), jnp.float32)
            acc8 = [z] * 8
            for j in range(CHUNK):
                x = buf[j, sl]
                t = j % 4
                acc8[2 * t] = acc8[2 * t] + plsc.bitcast(
                    lax.shift_left(x, 16), jnp.float32)
                acc8[2 * t + 1] = acc8[2 * t + 1] + plsc.bitcast(x, jnp.float32)
            ae = (acc8[0] + acc8[2]) + (acc8[4] + acc8[6])
            ao = (acc8[1] + acc8[3]) + (acc8[5] + acc8[7])
            if h == 0:
                pair_v[slot, 0, sl] = ae
                pair_v[slot, 1, sl] = ao
            elif h < NBUF - 1:
                pair_v[slot, 0, sl] = pair_v[slot, 0, sl] + ae
                pair_v[slot, 1, sl] = pair_v[slot, 1, sl] + ao
            else:
                acc_v[slot, sl] = (pair_v[slot, 0, sl] + ae) * (1.0 / SEQ)
                acc_v[slot, pl.ds(DIM2 + c * LANES, LANES)] = (
                    (pair_v[slot, 1, sl] + ao) * (1.0 / SEQ))
            return 0

        lax.fori_loop(0, GROUPS, group_body, 0)

    for p in range(BPW // HALF):
        prow = base + p * HALF
        pltpu.sync_copy(tok_hbm.at[pl.ds(prow, HALF)], tok_v)
        for h in range(NBUF):
            gstart(0, h, bufs[h], sems[h])

        def row_body(g, _):
            slot = lax.rem(g, 2)

            @pl.when(g >= 2)
            def _():
                pltpu.make_async_copy(acc_v.at[slot], out_hbm.at[prow], osem).wait()

            for h in range(NBUF):
                gwait(bufs[h], sems[h])
                accum(bufs[h], slot, h)

                @pl.when(g < HALF - 1)
                def _(h=h):
                    gstart(g + 1, h, bufs[h], sems[h])

            pltpu.make_async_copy(acc_v.at[slot], out_hbm.at[prow + g], osem).start()
            return 0

        lax.fori_loop(0, HALF, row_body, 0)
        pltpu.make_async_copy(acc_v.at[0], out_hbm.at[prow], osem).wait()
        pltpu.make_async_copy(acc_v.at[1], out_hbm.at[prow], osem).wait()


def _pool_entry(tok_hbm, ntab_hbm, out_hbm, tok_v, *rest):
    bufs, acc_v, pair_v, sems, osem = (
        rest[:NBUF], rest[NBUF], rest[NBUF + 1], rest[NBUF + 2:-1], rest[-1])
    _pool_body(tok_hbm, ntab_hbm, out_hbm, tok_v, bufs, acc_v, pair_v,
               sems, osem)


def _pooled_lookup(tokens, ntab_packed):
    mesh = plsc.VectorSubcoreMesh(core_axis_name="c", subcore_axis_name="s")
    run = functools.partial(
        pl.kernel, mesh=mesh,
        compiler_params=pltpu.CompilerParams(needs_layout_passes=False),
        out_type=jax.ShapeDtypeStruct((BATCH, DIM), jnp.float32),
        scratch_types=(
            [pltpu.VMEM((HALF, SEQ), jnp.int32)]
            + [pltpu.VMEM((CHUNK, DIMP2), jnp.int32) for _ in range(NBUF)]
            + [pltpu.VMEM((2, DIM), jnp.float32),
               pltpu.VMEM((2, 2, DIM2), jnp.float32)]
            + [pltpu.SemaphoreType.DMA for _ in range(NBUF + 1)]
        ),
    )(_pool_entry)
    return run(tokens, ntab_packed)


def kernel(tokens, embed_table, proj_w, norm_w):
    return _pooled_lookup(tokens, _projected_table(embed_table, proj_w, norm_w))


# speedup vs baseline: 1.0160x; 1.0160x over previous
import functools

import jax
import jax.numpy as jnp
from jax import lax
from jax.experimental import pallas as pl
from jax.experimental.pallas import tpu as pltpu
from jax.experimental.pallas import tpu_sc as plsc

VOCAB = 32128
DIM = 896
DIM2 = DIM // 2
DIMP2 = 512
DIMP = 2 * DIMP2
BATCH = 4096
SEQ = 128
EPS = 1.1920928955078125e-07

LANES = 16
NUM_WORKERS = 32
BPW = BATCH // NUM_WORKERS
HALF = BPW
CHUNK = 64
NBUF = SEQ // CHUNK
GROUPS = DIM2 // LANES


def _rne_bf16_bits(y):
    b = lax.bitcast_convert_type(y, jnp.int32)
    return lax.shift_right_logical(
        b + 0x7FFF + (lax.shift_right_logical(b, 16) & 1), 16)


def _proj_norm_body(nw_ref, emb_ref, w_ref, out_ref):
    x = emb_ref[...]
    y = lax.dot_general(x, w_ref[...], (((1,), (1,)), ((), ())),
                        preferred_element_type=jnp.float32)
    ms = jnp.mean(y * y, axis=1, keepdims=True)
    yn = y * lax.rsqrt(ms + EPS) * nw_ref[...]
    lo = _rne_bf16_bits(yn[:, :DIM2]) & 0xFFFF
    hi = lax.shift_left(_rne_bf16_bits(yn[:, DIM2:]), 16)
    out_ref[:, :DIM2] = lo | hi


def _projected_table(embed_table, proj_w, norm_w):
    blk = 1024
    return pl.pallas_call(
        _proj_norm_body,
        grid=(pl.cdiv(VOCAB, blk),),
        in_specs=[
            pl.BlockSpec((1, DIM), lambda i: (0, 0)),
            pl.BlockSpec((blk, DIM), lambda i: (i, 0)),
            pl.BlockSpec((DIM, DIM), lambda i: (0, 0)),
        ],
        out_specs=pl.BlockSpec((blk, DIMP2), lambda i: (i, 0)),
        out_shape=jax.ShapeDtypeStruct((VOCAB, DIMP2), jnp.int32),
    )(norm_w.reshape(1, DIM), embed_table, proj_w)


def _pool_body(tok_hbm, ntab_hbm, out_hbm,
               tok_v, bufs, acc_v, pair_v, sems, osem):
    wid = lax.axis_index("s") * 2 + lax.axis_index("c")
    base = wid * BPW

    def gstart(row, h, buf, sem):
        idx = tok_v.at[row, pl.ds(h * CHUNK, CHUNK)]
        pltpu.make_async_copy(ntab_hbm.at[idx], buf, sem).start()

    def gwait(buf, sem):
        idx = tok_v.at[0, pl.ds(0, CHUNK)]
        pltpu.make_async_copy(ntab_hbm.at[idx], buf, sem).wait()

    def accum(buf, slot, h):
        def group_body(c, _):
            sl = pl.ds(c * LANES, LANES)

            z = jnp.zeros((LANES,), jnp.float32)
            acc8 = [z] * 8
            for j in range(CHUNK):
                x = buf[j, sl]
                t = j % 4
                acc8[2 * t] = acc8[2 * t] + plsc.bitcast(
                    lax.shift_left(x, 16), jnp.float32)
                acc8[2 * t + 1] = acc8[2 * t + 1] + plsc.bitcast(x, jnp.float32)
            ae = (acc8[0] + acc8[2]) + (acc8[4] + acc8[6])
            ao = (acc8[1] + acc8[3]) + (acc8[5] + acc8[7])
            if h == 0:
                pair_v[slot, 0, sl] = ae
                pair_v[slot, 1, sl] = ao
            elif h < NBUF - 1:
                pair_v[slot, 0, sl] = pair_v[slot, 0, sl] + ae
                pair_v[slot, 1, sl] = pair_v[slot, 1, sl] + ao
            else:
                acc_v[slot, sl] = (pair_v[slot, 0, sl] + ae) * (1.0 / SEQ)
                acc_v[slot, pl.ds(DIM2 + c * LANES, LANES)] = (
                    (pair_v[slot, 1, sl] + ao) * (1.0 / SEQ))
            return 0

        lax.fori_loop(0, GROUPS, group_body, 0)

    for p in range(BPW // HALF):
        prow = base + p * HALF
        pltpu.sync_copy(tok_hbm.at[pl.ds(prow, HALF)], tok_v)
        for h in range(NBUF):
            gstart(0, h, bufs[h], sems[h])

        def row_body(g, _):
            slot = lax.rem(g, 2)

            @pl.when(g >= 2)
            def _():
                pltpu.make_async_copy(acc_v.at[slot], out_hbm.at[prow], osem).wait()

            for h in range(NBUF):
                gwait(bufs[h], sems[h])
                accum(bufs[h], slot, h)

                @pl.when(g < HALF - 1)
                def _(h=h):
                    gstart(g + 1, h, bufs[h], sems[h])

            pltpu.make_async_copy(acc_v.at[slot], out_hbm.at[prow + g], osem).start()
            return 0

        lax.fori_loop(0, HALF, row_body, 0)
        pltpu.make_async_copy(acc_v.at[0], out_hbm.at[prow], osem).wait()
        pltpu.make_async_copy(acc_v.at[1], out_hbm.at[prow], osem).wait()


def _pool_entry(tok_hbm, ntab_hbm, out_hbm, tok_v, *rest):
    bufs, acc_v, pair_v, sems, osem = (
        rest[:NBUF], rest[NBUF], rest[NBUF + 1], rest[NBUF + 2:-1], rest[-1])
    _pool_body(tok_hbm, ntab_hbm, out_hbm, tok_v, bufs, acc_v, pair_v,
               sems, osem)


def _pooled_lookup(tokens, ntab_packed):
    mesh = plsc.VectorSubcoreMesh(core_axis_name="c", subcore_axis_name="s")
    run = functools.partial(
        pl.kernel, mesh=mesh,
        compiler_params=pltpu.CompilerParams(needs_layout_passes=False),
        out_type=jax.ShapeDtypeStruct((BATCH, DIM), jnp.float32),
        scratch_types=(
            [pltpu.VMEM((HALF, SEQ), jnp.int32)]
            + [pltpu.VMEM((CHUNK, DIMP2), jnp.int32) for _ in range(NBUF)]
            + [pltpu.VMEM((2, DIM), jnp.float32),
               pltpu.VMEM((2, 2, DIM2), jnp.float32)]
            + [pltpu.SemaphoreType.DMA for _ in range(NBUF + 1)]
        ),
    )(_pool_entry)
    return run(tokens, ntab_packed)


def kernel(tokens, embed_table, proj_w, norm_w):
    return _pooled_lookup(tokens, _projected_table(embed_table, proj_w, norm_w))
